# baseline (speedup 1.0000x reference)
import functools

import jax
import jax.numpy as jnp
from jax import lax
from jax.experimental import pallas as pl
from jax.experimental.pallas import tpu as pltpu
from jax.experimental.pallas import tpu_sc as plsc

EPS = 1e-3

_NC = 2
_NS = 16
_NW = _NC * _NS

_G = 32
_L = 16


def _sc_gather(word_table, ids_flat, n_tokens, d):
  per_w = n_tokens // _NW
  ch = per_w // _G
  ids_resh = ids_flat.reshape(_NW, ch, _G)
  h = d // 2

  mesh = plsc.VectorSubcoreMesh(core_axis_name="c", subcore_axis_name="s")

  @functools.partial(
      pl.kernel,
      mesh=mesh,
      out_type=jax.ShapeDtypeStruct((n_tokens, h), jnp.int32),
      scratch_types=[
          pltpu.VMEM((ch, _G), jnp.int32),
          pltpu.VMEM((_G, d), jnp.float32),
          pltpu.VMEM((_G, d), jnp.float32),
          pltpu.VMEM((_G, h), jnp.int32),
          pltpu.VMEM((_G, h), jnp.int32),
          pltpu.SemaphoreType.DMA,
          pltpu.SemaphoreType.DMA,
          pltpu.SemaphoreType.DMA,
          pltpu.SemaphoreType.DMA,
      ],
  )
  def gather_kernel(table_hbm, ids_hbm, out_hbm, idx_v, rows0, rows1,
                    z0, z1, g0, g1, o0, o1):
    wid = lax.axis_index("s") * _NC + lax.axis_index("c")
    base = wid * per_w
    pltpu.sync_copy(ids_hbm.at[wid], idx_v)
    rows = (rows0, rows1)
    zb = (z0, z1)
    gsem = (g0, g1)
    osem = (o0, o1)

    def convert(rbuf, zbuf):
      @plsc.parallel_loop(0, _G, 1, unroll=4)
      def _(r):
        for k in range(h // _L):
          a = rbuf[r, pl.ds(k * _L, _L)]
          b2 = rbuf[r, pl.ds(h + k * _L, _L)]
          ai = lax.bitcast_convert_type(a, jnp.int32) + 0x8000
          bi = lax.bitcast_convert_type(b2, jnp.int32) + 0x8000
          za = ai & jnp.int32(-65536)
          zbits = lax.shift_right_logical(bi, 16)
          zbuf[r, pl.ds(k * _L, _L)] = za | zbits

    pltpu.async_copy(table_hbm.at[idx_v.at[0]], rows[0], gsem[0])
    pltpu.async_copy(table_hbm.at[idx_v.at[1]], rows[1], gsem[1])

    @pl.loop(0, ch, step=2)
    def _(c):
      for b in range(2):
        cc = c + b
        pltpu.make_async_copy(
            table_hbm.at[idx_v.at[cc]], rows[b], gsem[b]).wait()

        @pl.when(cc >= 2)
        def _():
          pltpu.make_async_copy(
              zb[b], out_hbm.at[pl.ds(base, _G)], osem[b]).wait()

        convert(rows[b], zb[b])
        pltpu.async_copy(
            zb[b], out_hbm.at[pl.ds(base + cc * _G, _G)], osem[b])

        @pl.when(cc + 2 < ch)
        def _():
          pltpu.async_copy(
              table_hbm.at[idx_v.at[cc + 2]], rows[b], gsem[b])

    for b in range(2):
      pltpu.make_async_copy(
          zb[b], out_hbm.at[pl.ds(base, _G)], osem[b]).wait()

  return gather_kernel(word_table, ids_resh)


def _ln_body(g_ref, tt_ref, ty_ref, pos_ref, gam_ref, bet_ref, out_ref):
  z = g_ref[0]
  xa = lax.bitcast_convert_type(z & jnp.int32(-65536), jnp.float32)
  xb = lax.bitcast_convert_type(z << 16, jnp.float32)
  x = jnp.concatenate([xa, xb], axis=-1)
  ttc = tt_ref[0]
  t0 = ty_ref[0:1, :]
  t1 = ty_ref[1:2, :]
  e = x + t0 + ttc * (t1 - t0) + pos_ref[...]
  m = jnp.mean(e, axis=-1, keepdims=True)
  dlt = e - m
  v = jnp.mean(dlt * dlt, axis=-1, keepdims=True)
  y = dlt * lax.rsqrt(v + EPS)
  out_ref[0] = y * gam_ref[...] + bet_ref[...]


def _ln_body_aliased(prev_ref, g_ref, tt_ref, ty_ref, pos_ref, gam_ref,
                     bet_ref, out_ref):
  del prev_ref
  _ln_body(g_ref, tt_ref, ty_ref, pos_ref, gam_ref, bet_ref, out_ref)


def _tc_ln_chunk(prev, gathered, ttf, type2, pos, gamma2d, beta2d,
                 b_total, base_blk):
  nb, s, hd = gathered.shape
  d = 2 * hd
  common_in = [
      pl.BlockSpec((1, s, hd), lambda i: (i, 0, 0)),
      pl.BlockSpec((1, s, 1), lambda i: (i, 0, 0)),
      pl.BlockSpec((2, d), lambda i: (0, 0)),
      pl.BlockSpec((s, d), lambda i: (0, 0)),
      pl.BlockSpec((1, d), lambda i: (0, 0)),
      pl.BlockSpec((1, d), lambda i: (0, 0)),
  ]
  out_spec = pl.BlockSpec((1, s, d), lambda i: (base_blk + i, 0, 0))
  out_shape = jax.ShapeDtypeStruct((b_total, s, d), jnp.float32)
  if prev is None:
    return pl.pallas_call(
        _ln_body,
        grid=(nb,),
        in_specs=common_in,
        out_specs=out_spec,
        out_shape=out_shape,
    )(gathered, ttf, type2, pos, gamma2d, beta2d)
  return pl.pallas_call(
      _ln_body_aliased,
      grid=(nb,),
      in_specs=[pl.BlockSpec(memory_space=pl.ANY)] + common_in,
      out_specs=out_spec,
      out_shape=out_shape,
      input_output_aliases={0: 0},
  )(prev, gathered, ttf, type2, pos, gamma2d, beta2d)


_K = 4


def kernel(inputs, token_type_ids, word_table, type_table, pos_table,
           ln_gamma, ln_beta):
  b, s = inputs.shape
  v, d = word_table.shape

  bc = b // _K
  n_tok_c = bc * s
  ids = inputs.reshape(_K, n_tok_c)
  ttf = token_type_ids.astype(jnp.float32).reshape(_K, bc, s, 1)
  type2 = type_table[:2]
  pos = pos_table[:s]
  gamma2d = ln_gamma.reshape(1, d)
  beta2d = ln_beta.reshape(1, d)

  out = None
  for k in range(_K):
    g_k = _sc_gather(word_table, ids[k], n_tok_c, d)
    out = _tc_ln_chunk(out, g_k.reshape(bc, s, d // 2), ttf[k], type2, pos,
                       gamma2d, beta2d, b, k * bc)
  return out


# speedup vs baseline: 1.0839x; 1.0839x over previous
import functools

import jax
import jax.numpy as jnp
from jax import lax
from jax.experimental import pallas as pl
from jax.experimental.pallas import tpu as pltpu
from jax.experimental.pallas import tpu_sc as plsc

EPS = 1e-3

_NC = 2
_NS = 16
_NW = _NC * _NS

_G = 32
_L = 16


def _sc_gather(word_table, ids_flat, n_tokens, d):
  per_w = n_tokens // _NW
  ch = per_w // _G
  ids_resh = ids_flat.reshape(_NW, ch, _G)
  h = d // 2

  mesh = plsc.VectorSubcoreMesh(core_axis_name="c", subcore_axis_name="s")

  @functools.partial(
      pl.kernel,
      mesh=mesh,
      out_type=jax.ShapeDtypeStruct((n_tokens, h), jnp.int32),
      scratch_types=[
          pltpu.VMEM((ch, _G), jnp.int32),
          pltpu.VMEM((_G, d), jnp.float32),
          pltpu.VMEM((_G, d), jnp.float32),
          pltpu.VMEM((_G, h), jnp.int32),
          pltpu.VMEM((_G, h), jnp.int32),
          pltpu.SemaphoreType.DMA,
          pltpu.SemaphoreType.DMA,
          pltpu.SemaphoreType.DMA,
          pltpu.SemaphoreType.DMA,
      ],
  )
  def gather_kernel(table_hbm, ids_hbm, out_hbm, idx_v, rows0, rows1,
                    z0, z1, g0, g1, o0, o1):
    wid = lax.axis_index("s") * _NC + lax.axis_index("c")
    base = wid * per_w
    pltpu.sync_copy(ids_hbm.at[wid], idx_v)
    rows = (rows0, rows1)
    zb = (z0, z1)
    gsem = (g0, g1)
    osem = (o0, o1)

    def convert(rbuf, zbuf):
      @plsc.parallel_loop(0, _G, 1, unroll=4)
      def _(r):
        for k in range(h // _L):
          a = rbuf[r, pl.ds(k * _L, _L)]
          b2 = rbuf[r, pl.ds(h + k * _L, _L)]
          ai = lax.bitcast_convert_type(a, jnp.int32) + 0x8000
          bi = lax.bitcast_convert_type(b2, jnp.int32) + 0x8000
          za = ai & jnp.int32(-65536)
          zbits = lax.shift_right_logical(bi, 16)
          zbuf[r, pl.ds(k * _L, _L)] = za | zbits

    pltpu.async_copy(table_hbm.at[idx_v.at[0]], rows[0], gsem[0])
    pltpu.async_copy(table_hbm.at[idx_v.at[1]], rows[1], gsem[1])

    @pl.loop(0, ch, step=2)
    def _(c):
      for b in range(2):
        cc = c + b
        pltpu.make_async_copy(
            table_hbm.at[idx_v.at[cc]], rows[b], gsem[b]).wait()

        @pl.when(cc >= 2)
        def _():
          pltpu.make_async_copy(
              zb[b], out_hbm.at[pl.ds(base, _G)], osem[b]).wait()

        convert(rows[b], zb[b])
        pltpu.async_copy(
            zb[b], out_hbm.at[pl.ds(base + cc * _G, _G)], osem[b])

        @pl.when(cc + 2 < ch)
        def _():
          pltpu.async_copy(
              table_hbm.at[idx_v.at[cc + 2]], rows[b], gsem[b])

    for b in range(2):
      pltpu.make_async_copy(
          zb[b], out_hbm.at[pl.ds(base, _G)], osem[b]).wait()

  return gather_kernel(word_table, ids_resh)


def _ln_body(g_ref, tt_ref, ty_ref, pos_ref, gam_ref, bet_ref, out_ref):
  z = g_ref[...]
  xa = lax.bitcast_convert_type(z & jnp.int32(-65536), jnp.float32)
  xb = lax.bitcast_convert_type(z << 16, jnp.float32)
  x = jnp.concatenate([xa, xb], axis=-1)
  ttc = tt_ref[...]
  t0 = ty_ref[0:1, :]
  t1 = ty_ref[1:2, :]
  e = x + t0 + ttc * (t1 - t0) + pos_ref[...]
  m = jnp.mean(e, axis=-1, keepdims=True)
  dlt = e - m
  v = jnp.mean(dlt * dlt, axis=-1, keepdims=True)
  y = dlt * lax.rsqrt(v + EPS)
  out_ref[...] = y * gam_ref[...] + bet_ref[...]


def _ln_body_aliased(prev_ref, g_ref, tt_ref, ty_ref, pos_ref, gam_ref,
                     bet_ref, out_ref):
  del prev_ref
  _ln_body(g_ref, tt_ref, ty_ref, pos_ref, gam_ref, bet_ref, out_ref)


def _tc_ln_chunk(prev, gathered, ttf, type2, pos, gamma2d, beta2d,
                 b_total, base_blk):
  nb, s, hd = gathered.shape
  d = 2 * hd
  bb = _BB
  nblk = nb // bb
  base = base_blk // bb
  common_in = [
      pl.BlockSpec((bb, s, hd), lambda i: (i, 0, 0)),
      pl.BlockSpec((bb, s, 1), lambda i: (i, 0, 0)),
      pl.BlockSpec((2, d), lambda i: (0, 0)),
      pl.BlockSpec((s, d), lambda i: (0, 0)),
      pl.BlockSpec((1, d), lambda i: (0, 0)),
      pl.BlockSpec((1, d), lambda i: (0, 0)),
  ]
  out_spec = pl.BlockSpec((bb, s, d), lambda i: (base + i, 0, 0))
  out_shape = jax.ShapeDtypeStruct((b_total, s, d), jnp.float32)
  if prev is None:
    return pl.pallas_call(
        _ln_body,
        grid=(nblk,),
        in_specs=common_in,
        out_specs=out_spec,
        out_shape=out_shape,
    )(gathered, ttf, type2, pos, gamma2d, beta2d)
  return pl.pallas_call(
      _ln_body_aliased,
      grid=(nblk,),
      in_specs=[pl.BlockSpec(memory_space=pl.ANY)] + common_in,
      out_specs=out_spec,
      out_shape=out_shape,
      input_output_aliases={0: 0},
  )(prev, gathered, ttf, type2, pos, gamma2d, beta2d)


_BB = 2
_K = 2


def kernel(inputs, token_type_ids, word_table, type_table, pos_table,
           ln_gamma, ln_beta):
  b, s = inputs.shape
  v, d = word_table.shape

  bc = b // _K
  n_tok_c = bc * s
  ids = inputs.reshape(_K, n_tok_c)
  ttf = token_type_ids.astype(jnp.float32).reshape(_K, bc, s, 1)
  type2 = type_table[:2]
  pos = pos_table[:s]
  gamma2d = ln_gamma.reshape(1, d)
  beta2d = ln_beta.reshape(1, d)

  out = None
  for k in range(_K):
    g_k = _sc_gather(word_table, ids[k], n_tok_c, d)
    out = _tc_ln_chunk(out, g_k.reshape(bc, s, d // 2), ttf[k], type2, pos,
                       gamma2d, beta2d, b, k * bc)
  return out


# speedup vs baseline: 1.1067x; 1.0210x over previous
import functools

import jax
import jax.numpy as jnp
from jax import lax
from jax.experimental import pallas as pl
from jax.experimental.pallas import tpu as pltpu
from jax.experimental.pallas import tpu_sc as plsc

EPS = 1e-3

_NC = 2
_NS = 16
_NW = _NC * _NS

_G = 32
_L = 16


def _sc_gather(word_table, ids_flat, n_tokens, d):
  per_w = n_tokens // _NW
  ch = per_w // _G
  ids_resh = ids_flat.reshape(_NW, ch, _G)
  h = d // 2

  mesh = plsc.VectorSubcoreMesh(core_axis_name="c", subcore_axis_name="s")

  @functools.partial(
      pl.kernel,
      mesh=mesh,
      out_type=jax.ShapeDtypeStruct((n_tokens, h), jnp.int32),
      scratch_types=[
          pltpu.VMEM((ch, _G), jnp.int32),
          pltpu.VMEM((_G, d), jnp.float32),
          pltpu.VMEM((_G, d), jnp.float32),
          pltpu.VMEM((_G, h), jnp.int32),
          pltpu.VMEM((_G, h), jnp.int32),
          pltpu.SemaphoreType.DMA,
          pltpu.SemaphoreType.DMA,
          pltpu.SemaphoreType.DMA,
          pltpu.SemaphoreType.DMA,
      ],
  )
  def gather_kernel(table_hbm, ids_hbm, out_hbm, idx_v, rows0, rows1,
                    z0, z1, g0, g1, o0, o1):
    wid = lax.axis_index("s") * _NC + lax.axis_index("c")
    base = wid * per_w
    pltpu.sync_copy(ids_hbm.at[wid], idx_v)
    rows = (rows0, rows1)
    zb = (z0, z1)
    gsem = (g0, g1)
    osem = (o0, o1)

    def convert(rbuf, zbuf):
      @plsc.parallel_loop(0, _G, 1, unroll=4)
      def _(r):
        for k in range(h // _L):
          a = rbuf[r, pl.ds(k * _L, _L)]
          b2 = rbuf[r, pl.ds(h + k * _L, _L)]
          ai = lax.bitcast_convert_type(a, jnp.int32) + 0x8000
          bi = lax.bitcast_convert_type(b2, jnp.int32) + 0x8000
          za = ai & jnp.int32(-65536)
          zbits = lax.shift_right_logical(bi, 16)
          zbuf[r, pl.ds(k * _L, _L)] = za | zbits

    pltpu.async_copy(table_hbm.at[idx_v.at[0]], rows[0], gsem[0])
    pltpu.async_copy(table_hbm.at[idx_v.at[1]], rows[1], gsem[1])

    @pl.loop(0, ch, step=2)
    def _(c):
      for b in range(2):
        cc = c + b
        pltpu.make_async_copy(
            table_hbm.at[idx_v.at[cc]], rows[b], gsem[b]).wait()

        @pl.when(cc >= 2)
        def _():
          pltpu.make_async_copy(
              zb[b], out_hbm.at[pl.ds(base, _G)], osem[b]).wait()

        convert(rows[b], zb[b])
        pltpu.async_copy(
            zb[b], out_hbm.at[pl.ds(base + cc * _G, _G)], osem[b])

        @pl.when(cc + 2 < ch)
        def _():
          pltpu.async_copy(
              table_hbm.at[idx_v.at[cc + 2]], rows[b], gsem[b])

    for b in range(2):
      pltpu.make_async_copy(
          zb[b], out_hbm.at[pl.ds(base, _G)], osem[b]).wait()

  return gather_kernel(word_table, ids_resh)


def _ln_body(g_ref, tt_ref, ty_ref, pos_ref, gam_ref, bet_ref, out_ref):
  z = g_ref[...]
  xa = lax.bitcast_convert_type(z & jnp.int32(-65536), jnp.float32)
  xb = lax.bitcast_convert_type(z << 16, jnp.float32)
  x = jnp.concatenate([xa, xb], axis=-1)
  ttc = tt_ref[...]
  t0 = ty_ref[0:1, :]
  t1 = ty_ref[1:2, :]
  e = x + t0 + ttc * (t1 - t0) + pos_ref[...]
  m = jnp.mean(e, axis=-1, keepdims=True)
  dlt = e - m
  v = jnp.mean(dlt * dlt, axis=-1, keepdims=True)
  y = dlt * lax.rsqrt(v + EPS)
  out_ref[...] = y * gam_ref[...] + bet_ref[...]


def _ln_body_aliased(prev_ref, g_ref, tt_ref, ty_ref, pos_ref, gam_ref,
                     bet_ref, out_ref):
  del prev_ref
  _ln_body(g_ref, tt_ref, ty_ref, pos_ref, gam_ref, bet_ref, out_ref)


def _tc_ln_chunk(prev, gathered, ttf, type2, pos, gamma2d, beta2d,
                 b_total, base_blk):
  nb, s, hd = gathered.shape
  d = 2 * hd
  bb = _BB
  nblk = nb // bb
  base = base_blk // bb
  common_in = [
      pl.BlockSpec((bb, s, hd), lambda i: (i, 0, 0)),
      pl.BlockSpec((bb, s, 1), lambda i: (i, 0, 0)),
      pl.BlockSpec((2, d), lambda i: (0, 0)),
      pl.BlockSpec((s, d), lambda i: (0, 0)),
      pl.BlockSpec((1, d), lambda i: (0, 0)),
      pl.BlockSpec((1, d), lambda i: (0, 0)),
  ]
  out_spec = pl.BlockSpec((bb, s, d), lambda i: (base + i, 0, 0))
  out_shape = jax.ShapeDtypeStruct((b_total, s, d), jnp.float32)
  if prev is None:
    return pl.pallas_call(
        _ln_body,
        grid=(nblk,),
        in_specs=common_in,
        out_specs=out_spec,
        out_shape=out_shape,
    )(gathered, ttf, type2, pos, gamma2d, beta2d)
  return pl.pallas_call(
      _ln_body_aliased,
      grid=(nblk,),
      in_specs=[pl.BlockSpec(memory_space=pl.ANY)] + common_in,
      out_specs=out_spec,
      out_shape=out_shape,
      input_output_aliases={0: 0},
  )(prev, gathered, ttf, type2, pos, gamma2d, beta2d)


_BB = 4
_K = 2


def kernel(inputs, token_type_ids, word_table, type_table, pos_table,
           ln_gamma, ln_beta):
  b, s = inputs.shape
  v, d = word_table.shape

  bc = b // _K
  n_tok_c = bc * s
  ids = inputs.reshape(_K, n_tok_c)
  ttf = token_type_ids.astype(jnp.float32).reshape(_K, bc, s, 1)
  type2 = type_table[:2]
  pos = pos_table[:s]
  gamma2d = ln_gamma.reshape(1, d)
  beta2d = ln_beta.reshape(1, d)

  out = None
  for k in range(_K):
    g_k = _sc_gather(word_table, ids[k], n_tok_c, d)
    out = _tc_ln_chunk(out, g_k.reshape(bc, s, d // 2), ttf[k], type2, pos,
                       gamma2d, beta2d, b, k * bc)
  return out
